# Initial kernel scaffold; baseline (speedup 1.0000x reference)
import functools

import jax
import jax.numpy as jnp
from jax import lax
from jax.experimental import pallas as pl
from jax.experimental.pallas import tpu as pltpu
from jax.experimental.pallas import tpu_sc as plsc

N = 10000
E = 320000
D = 128

NC = 2
NS = 16
L = 16
NW = NC * NS

B = 128
NBLK = 80
EPAD = NW * NBLK * B
ROWS_PER_TILE = N // NS

RB = 1000


def _k1_body(x_ref, w_ref, a_ref, xl_out, a_out):
    xl = jnp.dot(x_ref[...], w_ref[...], preferred_element_type=jnp.float32)
    xl_out[...] = xl
    a_out[...] = jnp.dot(xl, a_ref[...], preferred_element_type=jnp.float32)


def _tc_pre(x, W, A):
    return pl.pallas_call(
        _k1_body,
        grid=(N // RB,),
        in_specs=[
            pl.BlockSpec((RB, D), lambda i: (i, 0)),
            pl.BlockSpec((D, D), lambda i: (0, 0)),
            pl.BlockSpec((D, 8), lambda i: (0, 0)),
        ],
        out_specs=[
            pl.BlockSpec((RB, D), lambda i: (i, 0)),
            pl.BlockSpec((RB, 8), lambda i: (i, 0)),
        ],
        out_shape=[
            jax.ShapeDtypeStruct((N, D), jnp.float32),
            jax.ShapeDtypeStruct((N, 8), jnp.float32),
        ],
    )(x, W, A)


def _merge(accp, sp, xl, a, bias):
    acc = accp[0] + accp[1]
    sv = sp[0, :, 0:1] + sp[1, :, 0:1]
    e = a[:, 0:1] + a[:, 1:2]
    es = jnp.exp(jnp.where(e > 0, e, 0.2 * e))
    h = (acc + es * xl) / (sv + es + 1e-16) + bias
    return jnp.maximum(h, 0.0)


def _k2_body(accp_ref, sp_ref, xl_ref, a_ref, b_ref, w_ref, a2_ref,
             xl_out, a_out):
    h = _merge(accp_ref[...], sp_ref[...], xl_ref[...], a_ref[...], b_ref[...])
    xl2 = jnp.dot(h, w_ref[...], preferred_element_type=jnp.float32)
    xl_out[...] = xl2
    a_out[...] = jnp.dot(xl2, a2_ref[...], preferred_element_type=jnp.float32)


def _tc_mid(accp, sp, xl, a, bias, W2, A2):
    return pl.pallas_call(
        _k2_body,
        grid=(N // RB,),
        in_specs=[
            pl.BlockSpec((NC, RB, D), lambda i: (0, i, 0)),
            pl.BlockSpec((NC, RB, L), lambda i: (0, i, 0)),
            pl.BlockSpec((RB, D), lambda i: (i, 0)),
            pl.BlockSpec((RB, 8), lambda i: (i, 0)),
            pl.BlockSpec((1, D), lambda i: (0, 0)),
            pl.BlockSpec((D, D), lambda i: (0, 0)),
            pl.BlockSpec((D, 8), lambda i: (0, 0)),
        ],
        out_specs=[
            pl.BlockSpec((RB, D), lambda i: (i, 0)),
            pl.BlockSpec((RB, 8), lambda i: (i, 0)),
        ],
        out_shape=[
            jax.ShapeDtypeStruct((N, D), jnp.float32),
            jax.ShapeDtypeStruct((N, 8), jnp.float32),
        ],
    )(accp, sp, xl, a, bias, W2, A2)


def _k3_body(accp_ref, sp_ref, xl_ref, a_ref, b_ref,
             fc1w_ref, fc1b_ref, fc2w_ref, fc2b_ref, z_out):
    h = _merge(accp_ref[...], sp_ref[...], xl_ref[...], a_ref[...], b_ref[...])
    z = jnp.dot(h, fc1w_ref[...], preferred_element_type=jnp.float32)
    z = jnp.maximum(z + fc1b_ref[...], 0.0)
    z = jnp.dot(z, fc2w_ref[...], preferred_element_type=jnp.float32)
    z_out[...] = jnp.maximum(z + fc2b_ref[...], 0.0)


def _tc_post(accp, sp, xl, a, bias, fc1_w, fc1_b, fc2_w, fc2_b):
    return pl.pallas_call(
        _k3_body,
        grid=(N // RB,),
        in_specs=[
            pl.BlockSpec((NC, RB, D), lambda i: (0, i, 0)),
            pl.BlockSpec((NC, RB, L), lambda i: (0, i, 0)),
            pl.BlockSpec((RB, D), lambda i: (i, 0)),
            pl.BlockSpec((RB, 8), lambda i: (i, 0)),
            pl.BlockSpec((1, D), lambda i: (0, 0)),
            pl.BlockSpec((D, D), lambda i: (0, 0)),
            pl.BlockSpec((1, D), lambda i: (0, 0)),
            pl.BlockSpec((D, D), lambda i: (0, 0)),
            pl.BlockSpec((1, D), lambda i: (0, 0)),
        ],
        out_specs=pl.BlockSpec((RB, D), lambda i: (i, 0)),
        out_shape=jax.ShapeDtypeStruct((N, D), jnp.float32),
    )(accp, sp, xl, a, bias, fc1_w, fc1_b, fc2_w, fc2_b)


def _edge_body(srcb, dstb, asrc_h, adst_h, xl_h,
               accp_h, sp_h,
               asrc_v, adst_v, src_v, dst_v, ee_v,
               rows_v, srows_v,
               acc_sh, s_sh,
               gsem, ssem1, ssem2):
    c = lax.axis_index("c")
    s = lax.axis_index("s")
    w = c * NS + s

    pltpu.sync_copy(asrc_h, asrc_v)
    pltpu.sync_copy(adst_h, adst_v)
    pltpu.sync_copy(srcb.at[w], src_v)
    pltpu.sync_copy(dstb.at[w], dst_v)

    iota = lax.iota(jnp.int32, L)
    zeros = jnp.zeros((L,), jnp.float32)

    def _zrow(r, _):
        for k in range(D // L):
            rows_v[r, pl.ds(k * L, L)] = zeros
        srows_v[r, pl.ds(0, L)] = zeros
        return 0
    lax.fori_loop(0, 125, _zrow, 0)
    for i in range(5):
        base = s * ROWS_PER_TILE + i * 125
        pltpu.sync_copy(rows_v.at[pl.ds(0, 125)], acc_sh.at[pl.ds(base, 125)])
        pltpu.sync_copy(srows_v.at[pl.ds(0, 125)], s_sh.at[pl.ds(base, 125)])
    plsc.subcore_barrier()

    def _ee_blk(g, _):
        def _ee_grp(j, _):
            sv = src_v[g, pl.ds(j * L, L)]
            dv = dst_v[g, pl.ds(j * L, L)]
            ev = plsc.load_gather(asrc_v, [sv]) + plsc.load_gather(adst_v, [dv])
            ev = jnp.where(ev > 0, ev, 0.2 * ev)
            ee = jnp.exp(ev)
            eid = (w * NBLK + g) * B + j * L + iota
            ee_v[g, pl.ds(j * L, L)] = jnp.where(eid < E, ee, 0.0)
            return 0
        lax.fori_loop(0, B // L, _ee_grp, 0)
        return 0
    lax.fori_loop(0, NBLK, _ee_blk, 0)

    def _blk(g, _):
        pltpu.async_copy(xl_h.at[src_v.at[g]], rows_v, gsem).wait()

        def _scale(r, _):
            gi = jnp.full((L,), g, jnp.int32)
            ri = jnp.full((L,), r, jnp.int32)
            eev = plsc.load_gather(ee_v, [gi, ri])
            for k in range(D // L):
                rows_v[r, pl.ds(k * L, L)] = rows_v[r, pl.ds(k * L, L)] * eev
            srows_v[r, pl.ds(0, L)] = eev
            return 0
        lax.fori_loop(0, B, _scale, 0)

        d1 = pltpu.async_copy(rows_v, acc_sh.at[dst_v.at[g]], ssem1, add=True)
        d2 = pltpu.async_copy(srows_v, s_sh.at[dst_v.at[g]], ssem2, add=True)
        d1.wait()
        d2.wait()
        return 0
    lax.fori_loop(0, NBLK, _blk, 0)

    plsc.subcore_barrier()

    base = s * ROWS_PER_TILE
    pltpu.sync_copy(acc_sh.at[pl.ds(base, ROWS_PER_TILE)],
                    accp_h.at[c].at[pl.ds(base, ROWS_PER_TILE)])
    pltpu.sync_copy(s_sh.at[pl.ds(base, ROWS_PER_TILE)],
                    sp_h.at[c].at[pl.ds(base, ROWS_PER_TILE)])


@functools.partial(
    pl.kernel,
    out_type=[
        jax.ShapeDtypeStruct((NC, N, D), jnp.float32),
        jax.ShapeDtypeStruct((NC, N, L), jnp.float32),
    ],
    mesh=plsc.VectorSubcoreMesh(core_axis_name="c", subcore_axis_name="s"),
    scratch_types=[
        pltpu.VMEM((N,), jnp.float32),
        pltpu.VMEM((N,), jnp.float32),
        pltpu.VMEM((NBLK, B), jnp.int32),
        pltpu.VMEM((NBLK, B), jnp.int32),
        pltpu.VMEM((NBLK, B), jnp.float32),
        pltpu.VMEM((B, D), jnp.float32),
        pltpu.VMEM((B, L), jnp.float32),
        pltpu.VMEM_SHARED((N, D), jnp.float32),
        pltpu.VMEM_SHARED((N, L), jnp.float32),
        pltpu.SemaphoreType.DMA,
        pltpu.SemaphoreType.DMA,
        pltpu.SemaphoreType.DMA,
    ],
)
def _edge_pass(srcb, dstb, asrc, adst, xl, accp, sp, *scratch):
    _edge_body(srcb, dstb, asrc, adst, xl, accp, sp, *scratch)


def kernel(x, edge_index, W1, att_src1, att_dst1, bias1,
           W2, att_src2, att_dst2, bias2, fc1_w, fc1_b, fc2_w, fc2_b):
    pad = EPAD - E
    src = jnp.concatenate([edge_index[0], jnp.zeros((pad,), jnp.int32)])
    dst = jnp.concatenate([edge_index[1], jnp.zeros((pad,), jnp.int32)])
    srcb = src.reshape(NW, NBLK, B)
    dstb = dst.reshape(NW, NBLK, B)

    def att_mat(a_src, a_dst):
        A = jnp.zeros((D, 8), jnp.float32)
        return A.at[:, 0].set(a_src).at[:, 1].set(a_dst)

    A1 = att_mat(att_src1, att_dst1)
    A2 = att_mat(att_src2, att_dst2)

    xl1, a1 = _tc_pre(x, W1, A1)
    accp1, sp1 = _edge_pass(srcb, dstb, a1[:, 0], a1[:, 1], xl1)
    xl2, a2 = _tc_mid(accp1, sp1, xl1, a1, bias1.reshape(1, D), W2, A2)
    accp2, sp2 = _edge_pass(srcb, dstb, a2[:, 0], a2[:, 1], xl2)
    z = _tc_post(accp2, sp2, xl2, a2, bias2.reshape(1, D),
                 fc1_w, fc1_b.reshape(1, D), fc2_w, fc2_b.reshape(1, D))
    return z


# speedup vs baseline: 14.0086x; 14.0086x over previous
import functools

import jax
import jax.numpy as jnp
from jax import lax
from jax.experimental import pallas as pl
from jax.experimental.pallas import tpu as pltpu
from jax.experimental.pallas import tpu_sc as plsc

N = 10000
E = 320000
D = 128

NC = 2
NS = 16
L = 16
NW = NC * NS

B = 32
NBLK = 320
EPAD = NW * NBLK * B
RPT = 624

CH = 16
NT = N + 16

RB = 1000


def _k1_body(x_ref, w_ref, a_ref, xl_out, a_out):
    xl = jnp.dot(x_ref[...], w_ref[...], preferred_element_type=jnp.float32)
    xl_out[...] = xl
    a_out[...] = jnp.dot(xl, a_ref[...], preferred_element_type=jnp.float32)


def _tc_pre(x, W, A):
    return pl.pallas_call(
        _k1_body,
        grid=(N // RB,),
        in_specs=[
            pl.BlockSpec((RB, D), lambda i: (i, 0)),
            pl.BlockSpec((D, D), lambda i: (0, 0)),
            pl.BlockSpec((D, 8), lambda i: (0, 0)),
        ],
        out_specs=[
            pl.BlockSpec((RB, D), lambda i: (i, 0)),
            pl.BlockSpec((RB, 8), lambda i: (i, 0)),
        ],
        out_shape=[
            jax.ShapeDtypeStruct((N, D), jnp.float32),
            jax.ShapeDtypeStruct((N, 8), jnp.float32),
        ],
    )(x, W, A)


def _merge(acc0, acc1, s_all, xl, a, bias):
    acc = acc0 + acc1
    sv = jnp.sum(s_all[0], axis=0)[:, None]
    e = a[:, 0:1] + a[:, 1:2]
    es = jnp.exp(jnp.where(e > 0, e, 0.2 * e))
    h = (acc + es * xl) / (sv + es + 1e-16) + bias
    return jnp.maximum(h, 0.0)


def _k2_body(a0_ref, a1_ref, s_ref, xl_ref, a_ref, b_ref,
             w_ref, a2_ref, xl_out, a_out):
    h = _merge(a0_ref[...], a1_ref[...], s_ref[...],
               xl_ref[...], a_ref[...], b_ref[...])
    xl2 = jnp.dot(h, w_ref[...], preferred_element_type=jnp.float32)
    xl_out[...] = xl2
    a_out[...] = jnp.dot(xl2, a2_ref[...], preferred_element_type=jnp.float32)


def _tc_mid(acc0, acc1, s_all, xl, a, bias, W2, A2):
    return pl.pallas_call(
        _k2_body,
        grid=(N // RB,),
        in_specs=[
            pl.BlockSpec((RB, D), lambda i: (i, 0)),
            pl.BlockSpec((RB, D), lambda i: (i, 0)),
            pl.BlockSpec((1, NW, RB), lambda i: (i, 0, 0)),
            pl.BlockSpec((RB, D), lambda i: (i, 0)),
            pl.BlockSpec((RB, 8), lambda i: (i, 0)),
            pl.BlockSpec((1, D), lambda i: (0, 0)),
            pl.BlockSpec((D, D), lambda i: (0, 0)),
            pl.BlockSpec((D, 8), lambda i: (0, 0)),
        ],
        out_specs=[
            pl.BlockSpec((RB, D), lambda i: (i, 0)),
            pl.BlockSpec((RB, 8), lambda i: (i, 0)),
        ],
        out_shape=[
            jax.ShapeDtypeStruct((N, D), jnp.float32),
            jax.ShapeDtypeStruct((N, 8), jnp.float32),
        ],
    )(acc0, acc1, s_all, xl, a, bias, W2, A2)


def _k3_body(a0_ref, a1_ref, s_ref, xl_ref, a_ref, b_ref,
             fc1w_ref, fc1b_ref, fc2w_ref, fc2b_ref, z_out):
    h = _merge(a0_ref[...], a1_ref[...], s_ref[...],
               xl_ref[...], a_ref[...], b_ref[...])
    z = jnp.dot(h, fc1w_ref[...], preferred_element_type=jnp.float32)
    z = jnp.maximum(z + fc1b_ref[...], 0.0)
    z = jnp.dot(z, fc2w_ref[...], preferred_element_type=jnp.float32)
    z_out[...] = jnp.maximum(z + fc2b_ref[...], 0.0)


def _tc_post(acc0, acc1, s_all, xl, a, bias, fc1_w, fc1_b, fc2_w, fc2_b):
    return pl.pallas_call(
        _k3_body,
        grid=(N // RB,),
        in_specs=[
            pl.BlockSpec((RB, D), lambda i: (i, 0)),
            pl.BlockSpec((RB, D), lambda i: (i, 0)),
            pl.BlockSpec((1, NW, RB), lambda i: (i, 0, 0)),
            pl.BlockSpec((RB, D), lambda i: (i, 0)),
            pl.BlockSpec((RB, 8), lambda i: (i, 0)),
            pl.BlockSpec((1, D), lambda i: (0, 0)),
            pl.BlockSpec((D, D), lambda i: (0, 0)),
            pl.BlockSpec((1, D), lambda i: (0, 0)),
            pl.BlockSpec((D, D), lambda i: (0, 0)),
            pl.BlockSpec((1, D), lambda i: (0, 0)),
        ],
        out_specs=pl.BlockSpec((RB, D), lambda i: (i, 0)),
        out_shape=jax.ShapeDtypeStruct((N, D), jnp.float32),
    )(acc0, acc1, s_all, xl, a, bias, fc1_w, fc1_b, fc2_w, fc2_b)


WPT = 640


def _edge_body(srcb, dstb, asrc_h, adst_h, xl_h,
               accf_h, sf_h,
               asrc_v, adst_v, src_v, dst_v, ee_v,
               rows_v, s_loc,
               acc_sh,
               gsem, ssem1):
    c = lax.axis_index("c")
    s = lax.axis_index("s")
    w = c * NS + s

    zeros = jnp.zeros((L,), jnp.float32)

    pltpu.sync_copy(asrc_h, asrc_v)
    pltpu.sync_copy(adst_h, adst_v)

    def _zrow(r, _):
        for k in range(D // L):
            rows_v[r, pl.ds(k * L, L)] = zeros
        return 0
    lax.fori_loop(0, B, _zrow, 0)

    def _zs(i, _):
        s_loc[pl.ds(i * L, L)] = zeros
        return 0
    lax.fori_loop(0, NT // L, _zs, 0)

    def _zchunk(i, _):
        zbase = s * RPT + i * B
        pltpu.sync_copy(rows_v, acc_sh.at[pl.ds(zbase, B)])
        return 0
    lax.fori_loop(0, WPT // B, _zchunk, 0)

    plsc.subcore_barrier()

    def _chunk(ch, _):
        row0 = w * NBLK + ch * CH
        pltpu.sync_copy(srcb.at[pl.ds(row0, CH)], src_v)
        pltpu.sync_copy(dstb.at[pl.ds(row0, CH)], dst_v)

        def _blk(g, _):
            d_rows = pltpu.async_copy(xl_h.at[src_v.at[g]], rows_v, gsem)

            def _ee_grp(j, _):
                sv = src_v[g, pl.ds(j * L, L)]
                dv = dst_v[g, pl.ds(j * L, L)]
                ev = (plsc.load_gather(asrc_v, [sv])
                      + plsc.load_gather(adst_v, [dv]))
                ev = jnp.where(ev > 0, ev, 0.2 * ev)
                ee = jnp.exp(ev)
                ee_v[pl.ds(j * L, L)] = ee
                plsc.addupdate_scatter(s_loc, [dv], ee)
                return 0
            lax.fori_loop(0, B // L, _ee_grp, 0)

            d_rows.wait()

            def _scale(r, _):
                ri = jnp.full((L,), r, jnp.int32)
                eev = plsc.load_gather(ee_v, [ri])
                for k in range(D // L):
                    rows_v[r, pl.ds(k * L, L)] = (
                        rows_v[r, pl.ds(k * L, L)] * eev)
                return 0
            lax.fori_loop(0, B, _scale, 0)

            pltpu.async_copy(rows_v, acc_sh.at[dst_v.at[g]], ssem1,
                             add=True).wait()
            return 0
        lax.fori_loop(0, CH, _blk, 0)
        return 0
    lax.fori_loop(0, NBLK // CH, _chunk, 0)

    plsc.subcore_barrier()

    base = s * RPT
    pltpu.sync_copy(acc_sh.at[pl.ds(base, WPT)],
                    accf_h.at[pl.ds(c * N + base, WPT)])
    pltpu.sync_copy(s_loc, sf_h.at[pl.ds(w * NT, NT)])


@functools.partial(
    pl.kernel,
    out_type=[
        jax.ShapeDtypeStruct((NC * N, D), jnp.float32),
        jax.ShapeDtypeStruct((NW * NT,), jnp.float32),
    ],
    mesh=plsc.VectorSubcoreMesh(core_axis_name="c", subcore_axis_name="s"),
    compiler_params=pltpu.CompilerParams(needs_layout_passes=False),
    scratch_types=[
        pltpu.VMEM((NT,), jnp.float32),
        pltpu.VMEM((NT,), jnp.float32),
        pltpu.VMEM((CH, B), jnp.int32),
        pltpu.VMEM((CH, B), jnp.int32),
        pltpu.VMEM((B,), jnp.float32),
        pltpu.VMEM((B, D), jnp.float32),
        pltpu.VMEM((NT,), jnp.float32),
        pltpu.VMEM_SHARED((NT, D), jnp.float32),
        pltpu.SemaphoreType.DMA,
        pltpu.SemaphoreType.DMA,
    ],
)
def _edge_pass(srcb, dstb, asrc, adst, xl, accf, sf, *scratch):
    _edge_body(srcb, dstb, asrc, adst, xl, accf, sf, *scratch)


def kernel(x, edge_index, W1, att_src1, att_dst1, bias1,
           W2, att_src2, att_dst2, bias2, fc1_w, fc1_b, fc2_w, fc2_b):
    pad = EPAD - E
    src = jnp.concatenate([edge_index[0], jnp.zeros((pad,), jnp.int32)])
    dst = jnp.concatenate([edge_index[1],
                           jnp.full((pad,), N, jnp.int32)])
    srcb = src.reshape(NW * NBLK, B)
    dstb = dst.reshape(NW * NBLK, B)

    def att_mat(a_src, a_dst):
        A = jnp.zeros((D, 8), jnp.float32)
        return A.at[:, 0].set(a_src).at[:, 1].set(a_dst)

    A1 = att_mat(att_src1, att_dst1)
    A2 = att_mat(att_src2, att_dst2)

    def padt(v):
        return jnp.concatenate([v, jnp.zeros((NT - N,), jnp.float32)])

    xl1, a1 = _tc_pre(x, W1, A1)
    accf1, sf1 = _edge_pass(srcb, dstb, padt(a1[:, 0]), padt(a1[:, 1]), xl1)
    s1_all = sf1.reshape(NW, NT)[:, :N].reshape(NW, N // RB, RB).swapaxes(0, 1)
    xl2, a2 = _tc_mid(accf1[:N], accf1[N:], s1_all,
                      xl1, a1, bias1.reshape(1, D), W2, A2)
    accf2, sf2 = _edge_pass(srcb, dstb, padt(a2[:, 0]), padt(a2[:, 1]), xl2)
    s2_all = sf2.reshape(NW, NT)[:, :N].reshape(NW, N // RB, RB).swapaxes(0, 1)
    z = _tc_post(accf2[:N], accf2[N:], s2_all,
                 xl2, a2, bias2.reshape(1, D),
                 fc1_w, fc1_b.reshape(1, D), fc2_w, fc2_b.reshape(1, D))
    return z


# speedup vs baseline: 15.3467x; 1.0955x over previous
import functools

import jax
import jax.numpy as jnp
from jax import lax
from jax.experimental import pallas as pl
from jax.experimental.pallas import tpu as pltpu
from jax.experimental.pallas import tpu_sc as plsc

N = 10000
E = 320000
D = 128

NC = 2
NS = 16
L = 16
NW = NC * NS

B = 32
NBLK = 320
EPAD = NW * NBLK * B
RPT = 624

CH = 8
NT = N + 16

RB = 1000


def _k1_body(x_ref, w_ref, a_ref, xl_out, a_out):
    xl = jnp.dot(x_ref[...], w_ref[...], preferred_element_type=jnp.float32)
    xl_out[...] = xl
    a_out[...] = jnp.dot(xl, a_ref[...], preferred_element_type=jnp.float32)


def _tc_pre(x, W, A):
    return pl.pallas_call(
        _k1_body,
        grid=(N // RB,),
        in_specs=[
            pl.BlockSpec((RB, D), lambda i: (i, 0)),
            pl.BlockSpec((D, D), lambda i: (0, 0)),
            pl.BlockSpec((D, 8), lambda i: (0, 0)),
        ],
        out_specs=[
            pl.BlockSpec((RB, D), lambda i: (i, 0)),
            pl.BlockSpec((RB, 8), lambda i: (i, 0)),
        ],
        out_shape=[
            jax.ShapeDtypeStruct((N, D), jnp.float32),
            jax.ShapeDtypeStruct((N, 8), jnp.float32),
        ],
    )(x, W, A)


def _merge(acc0, acc1, s_all, xl, a, bias):
    acc = acc0 + acc1
    sv = jnp.sum(s_all[0], axis=0)[:, None]
    e = a[:, 0:1] + a[:, 1:2]
    es = jnp.exp(jnp.where(e > 0, e, 0.2 * e))
    h = (acc + es * xl) / (sv + es + 1e-16) + bias
    return jnp.maximum(h, 0.0)


def _k2_body(a0_ref, a1_ref, s_ref, xl_ref, a_ref, b_ref,
             w_ref, a2_ref, xl_out, a_out):
    h = _merge(a0_ref[...], a1_ref[...], s_ref[...],
               xl_ref[...], a_ref[...], b_ref[...])
    xl2 = jnp.dot(h, w_ref[...], preferred_element_type=jnp.float32)
    xl_out[...] = xl2
    a_out[...] = jnp.dot(xl2, a2_ref[...], preferred_element_type=jnp.float32)


def _tc_mid(acc0, acc1, s_all, xl, a, bias, W2, A2):
    return pl.pallas_call(
        _k2_body,
        grid=(N // RB,),
        in_specs=[
            pl.BlockSpec((RB, D), lambda i: (i, 0)),
            pl.BlockSpec((RB, D), lambda i: (i, 0)),
            pl.BlockSpec((1, NW, RB), lambda i: (i, 0, 0)),
            pl.BlockSpec((RB, D), lambda i: (i, 0)),
            pl.BlockSpec((RB, 8), lambda i: (i, 0)),
            pl.BlockSpec((1, D), lambda i: (0, 0)),
            pl.BlockSpec((D, D), lambda i: (0, 0)),
            pl.BlockSpec((D, 8), lambda i: (0, 0)),
        ],
        out_specs=[
            pl.BlockSpec((RB, D), lambda i: (i, 0)),
            pl.BlockSpec((RB, 8), lambda i: (i, 0)),
        ],
        out_shape=[
            jax.ShapeDtypeStruct((N, D), jnp.float32),
            jax.ShapeDtypeStruct((N, 8), jnp.float32),
        ],
    )(acc0, acc1, s_all, xl, a, bias, W2, A2)


def _k3_body(a0_ref, a1_ref, s_ref, xl_ref, a_ref, b_ref,
             fc1w_ref, fc1b_ref, fc2w_ref, fc2b_ref, z_out):
    h = _merge(a0_ref[...], a1_ref[...], s_ref[...],
               xl_ref[...], a_ref[...], b_ref[...])
    z = jnp.dot(h, fc1w_ref[...], preferred_element_type=jnp.float32)
    z = jnp.maximum(z + fc1b_ref[...], 0.0)
    z = jnp.dot(z, fc2w_ref[...], preferred_element_type=jnp.float32)
    z_out[...] = jnp.maximum(z + fc2b_ref[...], 0.0)


def _tc_post(acc0, acc1, s_all, xl, a, bias, fc1_w, fc1_b, fc2_w, fc2_b):
    return pl.pallas_call(
        _k3_body,
        grid=(N // RB,),
        in_specs=[
            pl.BlockSpec((RB, D), lambda i: (i, 0)),
            pl.BlockSpec((RB, D), lambda i: (i, 0)),
            pl.BlockSpec((1, NW, RB), lambda i: (i, 0, 0)),
            pl.BlockSpec((RB, D), lambda i: (i, 0)),
            pl.BlockSpec((RB, 8), lambda i: (i, 0)),
            pl.BlockSpec((1, D), lambda i: (0, 0)),
            pl.BlockSpec((D, D), lambda i: (0, 0)),
            pl.BlockSpec((1, D), lambda i: (0, 0)),
            pl.BlockSpec((D, D), lambda i: (0, 0)),
            pl.BlockSpec((1, D), lambda i: (0, 0)),
        ],
        out_specs=pl.BlockSpec((RB, D), lambda i: (i, 0)),
        out_shape=jax.ShapeDtypeStruct((N, D), jnp.float32),
    )(acc0, acc1, s_all, xl, a, bias, fc1_w, fc1_b, fc2_w, fc2_b)


WPT = 640


def _edge_body(srcb, dstb, asrc_h, adst_h, xl_h,
               accf_h, sf_h,
               asrc_v, adst_v, src_v, dst_v,
               ee0_v, ee1_v, rows0_v, rows1_v, s_loc,
               acc_sh,
               gsem0, gsem1, ssem0, ssem1):
    c = lax.axis_index("c")
    s = lax.axis_index("s")
    w = c * NS + s

    zeros = jnp.zeros((L,), jnp.float32)

    pltpu.sync_copy(asrc_h, asrc_v)
    pltpu.sync_copy(adst_h, adst_v)

    def _zrow(r, _):
        for k in range(D // L):
            rows0_v[r, pl.ds(k * L, L)] = zeros
        return 0
    lax.fori_loop(0, B, _zrow, 0)

    def _zs(i, _):
        s_loc[pl.ds(i * L, L)] = zeros
        return 0
    lax.fori_loop(0, NT // L, _zs, 0)

    def _zchunk(i, _):
        zbase = s * RPT + i * B
        pltpu.sync_copy(rows0_v, acc_sh.at[pl.ds(zbase, B)])
        return 0
    lax.fori_loop(0, WPT // B, _zchunk, 0)

    plsc.subcore_barrier()

    def _ee(g, ee_v):
        def _ee_grp(j, _):
            sv = src_v[g, pl.ds(j * L, L)]
            dv = dst_v[g, pl.ds(j * L, L)]
            ev = (plsc.load_gather(asrc_v, [sv])
                  + plsc.load_gather(adst_v, [dv]))
            ev = jnp.where(ev > 0, ev, 0.2 * ev)
            ee = jnp.exp(ev)
            ee_v[pl.ds(j * L, L)] = ee
            plsc.addupdate_scatter(s_loc, [dv], ee)
            return 0
        lax.fori_loop(0, B // L, _ee_grp, 0)

    def _scale(rows_v, ee_v):
        def _srow(r, _):
            ri = jnp.full((L,), r, jnp.int32)
            eev = plsc.load_gather(ee_v, [ri])
            for k in range(D // L):
                rows_v[r, pl.ds(k * L, L)] = rows_v[r, pl.ds(k * L, L)] * eev
            return 0
        lax.fori_loop(0, B, _srow, 0)

    def _drain():
        pltpu.make_async_copy(rows0_v, acc_sh.at[dst_v.at[0]], ssem0).wait()
        pltpu.make_async_copy(rows1_v, acc_sh.at[dst_v.at[0]], ssem1).wait()

    def _pair(g0, g1, drain_first):
        if drain_first:
            _drain()
        d0 = pltpu.async_copy(xl_h.at[src_v.at[g0]], rows0_v, gsem0)
        d1 = pltpu.async_copy(xl_h.at[src_v.at[g1]], rows1_v, gsem1)
        _ee(g0, ee0_v)
        d0.wait()
        _scale(rows0_v, ee0_v)
        pltpu.async_copy(rows0_v, acc_sh.at[dst_v.at[g0]], ssem0, add=True)
        _ee(g1, ee1_v)
        d1.wait()
        _scale(rows1_v, ee1_v)
        pltpu.async_copy(rows1_v, acc_sh.at[dst_v.at[g1]], ssem1, add=True)

    def _chunk(ch, _):
        row0 = w * NBLK + ch * CH
        pltpu.sync_copy(srcb.at[pl.ds(row0, CH)], src_v)
        pltpu.sync_copy(dstb.at[pl.ds(row0, CH)], dst_v)

        _pair(0, 1, False)

        def _body(g2, _):
            _pair(2 * g2, 2 * g2 + 1, True)
            return 0
        lax.fori_loop(1, CH // 2, _body, 0)
        _drain()
        return 0
    lax.fori_loop(0, NBLK // CH, _chunk, 0)

    plsc.subcore_barrier()

    base = s * RPT
    pltpu.sync_copy(acc_sh.at[pl.ds(base, WPT)],
                    accf_h.at[pl.ds(c * N + base, WPT)])
    pltpu.sync_copy(s_loc, sf_h.at[pl.ds(w * NT, NT)])


@functools.partial(
    pl.kernel,
    out_type=[
        jax.ShapeDtypeStruct((NC * N, D), jnp.float32),
        jax.ShapeDtypeStruct((NW * NT,), jnp.float32),
    ],
    mesh=plsc.VectorSubcoreMesh(core_axis_name="c", subcore_axis_name="s"),
    compiler_params=pltpu.CompilerParams(needs_layout_passes=False),
    scratch_types=[
        pltpu.VMEM((NT,), jnp.float32),
        pltpu.VMEM((NT,), jnp.float32),
        pltpu.VMEM((CH, B), jnp.int32),
        pltpu.VMEM((CH, B), jnp.int32),
        pltpu.VMEM((B,), jnp.float32),
        pltpu.VMEM((B,), jnp.float32),
        pltpu.VMEM((B, D), jnp.float32),
        pltpu.VMEM((B, D), jnp.float32),
        pltpu.VMEM((NT,), jnp.float32),
        pltpu.VMEM_SHARED((NT, D), jnp.float32),
        pltpu.SemaphoreType.DMA,
        pltpu.SemaphoreType.DMA,
        pltpu.SemaphoreType.DMA,
        pltpu.SemaphoreType.DMA,
    ],
)
def _edge_pass(srcb, dstb, asrc, adst, xl, accf, sf, *scratch):
    _edge_body(srcb, dstb, asrc, adst, xl, accf, sf, *scratch)


def kernel(x, edge_index, W1, att_src1, att_dst1, bias1,
           W2, att_src2, att_dst2, bias2, fc1_w, fc1_b, fc2_w, fc2_b):
    pad = EPAD - E
    src = jnp.concatenate([edge_index[0], jnp.zeros((pad,), jnp.int32)])
    dst = jnp.concatenate([edge_index[1],
                           jnp.full((pad,), N, jnp.int32)])
    srcb = src.reshape(NW * NBLK, B)
    dstb = dst.reshape(NW * NBLK, B)

    def att_mat(a_src, a_dst):
        A = jnp.zeros((D, 8), jnp.float32)
        return A.at[:, 0].set(a_src).at[:, 1].set(a_dst)

    A1 = att_mat(att_src1, att_dst1)
    A2 = att_mat(att_src2, att_dst2)

    def padt(v):
        return jnp.concatenate([v, jnp.zeros((NT - N,), jnp.float32)])

    xl1, a1 = _tc_pre(x, W1, A1)
    accf1, sf1 = _edge_pass(srcb, dstb, padt(a1[:, 0]), padt(a1[:, 1]), xl1)
    s1_all = sf1.reshape(NW, NT)[:, :N].reshape(NW, N // RB, RB).swapaxes(0, 1)
    xl2, a2 = _tc_mid(accf1[:N], accf1[N:], s1_all,
                      xl1, a1, bias1.reshape(1, D), W2, A2)
    accf2, sf2 = _edge_pass(srcb, dstb, padt(a2[:, 0]), padt(a2[:, 1]), xl2)
    s2_all = sf2.reshape(NW, NT)[:, :N].reshape(NW, N // RB, RB).swapaxes(0, 1)
    z = _tc_post(accf2[:N], accf2[N:], s2_all,
                 xl2, a2, bias2.reshape(1, D),
                 fc1_w, fc1_b.reshape(1, D), fc2_w, fc2_b.reshape(1, D))
    return z
